# baseline (speedup 1.0000x reference)
import jax
import jax.numpy as jnp
from jax import lax
from jax.experimental import pallas as pl
from jax.experimental.pallas import tpu as pltpu
from jax.experimental.pallas import tpu_sc as plsc

N = 10000
E = 320000
IN = 128
HEADS = 8
HD = 8
HID = 64
NC = 10
NG = 16
SIGMA = 0.5

NP = 10240
BKT = 320
PAD_NODE = 10016
CAP = 12800
SCAN_C = 2000
C = 512

_SC_PARAMS = pltpu.CompilerParams(needs_layout_passes=False,
                                  use_tc_tiling_on_sc=False)


def _i16():
    return lax.broadcasted_iota(jnp.int32, (16,), 0)


def _f16(v):
    return jnp.full((16,), v, jnp.int32)


def _wid():
    return lax.axis_index("c") * 16 + lax.axis_index("s")


def _pre_call():
    def body(src_r, dst_r, srcq_o, dstq_o, cnts_o, sv, dv, sq, dq, cw):
        wid = _wid()
        lo = wid * BKT
        hi = lo + BKT

        def pf(i, c):
            o = pl.multiple_of(i * 16, 16)
            sq[pl.ds(o, 16)] = _f16(PAD_NODE)
            dq[pl.ds(o, 16)] = _f16(PAD_NODE)
            return c

        lax.fori_loop(0, (CAP + 16) // 16, pf, 0)

        def outer(ch, pos):
            pltpu.sync_copy(src_r.at[pl.ds(ch * SCAN_C, SCAN_C)], sv)
            pltpu.sync_copy(dst_r.at[pl.ds(ch * SCAN_C, SCAN_C)], dv)

            def inner(j, p):
                o = pl.multiple_of(j * 16, 16)
                s16 = sv[pl.ds(o, 16)]
                d16 = dv[pl.ds(o, 16)]
                m = (d16 >= lo) & (d16 < hi)
                plsc.store_compressed(sq.at[pl.ds(p, 16)], s16, mask=m)
                plsc.store_compressed(dq.at[pl.ds(p, 16)], d16, mask=m)
                pc = plsc.all_reduce_population_count(m)
                return p + pc[0]

            return lax.fori_loop(0, SCAN_C // 16, inner, pos)

        pos = lax.fori_loop(0, E // SCAN_C, outer, 0)
        pltpu.sync_copy(sq.at[pl.ds(0, CAP)], srcq_o.at[wid])
        pltpu.sync_copy(dq.at[pl.ds(0, CAP)], dstq_o.at[wid])
        cw[...] = jnp.zeros((16,), jnp.int32) + pos
        pltpu.sync_copy(cw, cnts_o.at[wid])

    return pl.kernel(
        body,
        out_type=(jax.ShapeDtypeStruct((32, CAP), jnp.int32),
                  jax.ShapeDtypeStruct((32, CAP), jnp.int32),
                  jax.ShapeDtypeStruct((32, 16), jnp.int32)),
        mesh=plsc.VectorSubcoreMesh(core_axis_name="c", subcore_axis_name="s"),
        scratch_types=(pltpu.VMEM((SCAN_C,), jnp.int32),
                       pltpu.VMEM((SCAN_C,), jnp.int32),
                       pltpu.VMEM((CAP + 16,), jnp.int32),
                       pltpu.VMEM((CAP + 16,), jnp.int32),
                       pltpu.VMEM((16,), jnp.int32)),
        compiler_params=_SC_PARAMS,
    )


def _layer_call(heads, tw, compute_ew):
    fan = 64 // heads
    out_type = [jax.ShapeDtypeStruct((NP * 64,), jnp.float32),
                jax.ShapeDtypeStruct((NP * 16,), jnp.float32)]
    if compute_ew:
        out_type.append(jax.ShapeDtypeStruct((32, CAP), jnp.float32))
    scratch = [
        pltpu.VMEM((C,), jnp.int32),
        pltpu.VMEM((C,), jnp.int32),
        pltpu.VMEM((C, tw), jnp.float32),
        pltpu.VMEM((C, tw), jnp.float32),
        pltpu.VMEM((C, 64), jnp.float32),
        pltpu.VMEM((C,), jnp.float32),
        pltpu.VMEM((heads, 16), jnp.float32),
        pltpu.VMEM((16,), jnp.int32),
        pltpu.VMEM((BKT * 64,), jnp.float32),
        pltpu.VMEM((BKT * 16,), jnp.float32),
        pltpu.SemaphoreType.DMA,
    ]

    def body(*refs):
        if compute_ew:
            (srcq_r, dstq_r, cnts_r, t_r, h_r, ce_r,
             acc_o, den_o, ewq_o,
             idxs, idxd, ts, td, hv, ewv, cevv, cw, accT, denT, sem) = refs
        else:
            (srcq_r, dstq_r, cnts_r, t_r, h_r, ce_r, ewq_r,
             acc_o, den_o,
             idxs, idxd, ts, td, hv, ewv, cevv, cw, accT, denT, sem) = refs
        wid = _wid()
        lo = wid * BKT

        def z64(i, c):
            o = pl.multiple_of(i * 16, 16)
            accT[pl.ds(o, 16)] = jnp.zeros((16,), jnp.float32)
            return c

        def z16(i, c):
            o = pl.multiple_of(i * 16, 16)
            denT[pl.ds(o, 16)] = jnp.zeros((16,), jnp.float32)
            return c

        lax.fori_loop(0, BKT * 4, z64, 0)
        lax.fori_loop(0, BKT, z16, 0)
        pltpu.sync_copy(ce_r, cevv)
        pltpu.sync_copy(cnts_r.at[wid], cw)
        cnt = jnp.max(cw[...])
        nch = (cnt + (C - 1)) // C
        ce_b = [cevv[h] for h in range(heads)]
        i16 = _i16()

        def chunk(ci, carry):
            off = ci * C
            pltpu.sync_copy(srcq_r.at[wid, pl.ds(off, C)], idxs)
            pltpu.sync_copy(dstq_r.at[wid, pl.ds(off, C)], idxd)
            if not compute_ew:
                pltpu.sync_copy(ewq_r.at[wid, pl.ds(off, C)], ewv)
            descs = []
            for b in range(C // 128):
                sl = pl.ds(b * 128, 128)
                descs.append(pltpu.async_copy(
                    t_r.at[idxs.at[sl]], ts.at[sl], sem))
                descs.append(pltpu.async_copy(
                    t_r.at[idxd.at[sl]], td.at[sl], sem))
                descs.append(pltpu.async_copy(
                    h_r.at[idxs.at[sl]], hv.at[sl], sem))
            for d in descs:
                d.wait()

            def jb(j, c2):
                o = pl.multiple_of(j * 16, 16)
                rows = j * 16 + i16
                msk = (off + rows) < cnt
                lr = idxd[pl.ds(o, 16)] - lo
                if compute_ew:
                    dacc = None
                    for k in range(3):
                        kf = _f16(16 + k)
                        df = (plsc.load_gather(ts, [rows, kf])
                              - plsc.load_gather(td, [rows, kf]))
                        dacc = df * df if dacc is None else dacc + df * df
                    ew16 = jnp.exp(dacc * (-1.0 / (2.0 * SIGMA * SIGMA)))
                    ewv[pl.ds(o, 16)] = ew16
                else:
                    ew16 = ewv[pl.ds(o, 16)]
                for h in range(heads):
                    a = (plsc.load_gather(ts, [rows, _f16(h)])
                         + plsc.load_gather(td, [rows, _f16(h + 8)])
                         + ce_b[h] * ew16)
                    a = jnp.maximum(a, a * 0.2)
                    ex = jnp.exp(a)
                    plsc.addupdate_scatter(denT, [lr * 16 + h], ex, mask=msk)
                    for dd in range(fan):
                        col = h * fan + dd
                        v = plsc.load_gather(hv, [rows, _f16(col)]) * ex
                        plsc.addupdate_scatter(accT, [lr * 64 + col], v,
                                               mask=msk)
                return c2

            lax.fori_loop(0, C // 16, jb, 0)
            if compute_ew:
                pltpu.sync_copy(ewv, ewq_o.at[wid, pl.ds(off, C)])
            return carry

        lax.fori_loop(0, nch, chunk, 0)
        pltpu.sync_copy(accT, acc_o.at[pl.ds(lo * 64, BKT * 64)])
        pltpu.sync_copy(denT, den_o.at[pl.ds(lo * 16, BKT * 16)])

    return pl.kernel(
        body,
        out_type=tuple(out_type),
        mesh=plsc.VectorSubcoreMesh(core_axis_name="c", subcore_axis_name="s"),
        scratch_types=tuple(scratch),
        compiler_params=_SC_PARAMS,
    )


def _prep_body(x_ref, w_ref, a_ref, rx_ref, h_ref, t_ref):
    h = jnp.dot(x_ref[...], w_ref[...], preferred_element_type=jnp.float32)
    h_ref[...] = h
    al = jnp.dot(h, a_ref[...], preferred_element_type=jnp.float32)
    t_ref[...] = jnp.concatenate([al, rx_ref[...]], axis=1)


def _tc_prep(xp, W, A, rxp8):
    BLK = 2048
    return pl.pallas_call(
        _prep_body,
        grid=(NP // BLK,),
        in_specs=[
            pl.BlockSpec((BLK, IN), lambda i: (i, 0)),
            pl.BlockSpec((IN, 64), lambda i: (0, 0)),
            pl.BlockSpec((64, 16), lambda i: (0, 0)),
            pl.BlockSpec((BLK, 8), lambda i: (i, 0)),
        ],
        out_specs=[
            pl.BlockSpec((BLK, 64), lambda i: (i, 0)),
            pl.BlockSpec((BLK, 24), lambda i: (i, 0)),
        ],
        out_shape=[
            jax.ShapeDtypeStruct((NP, 64), jnp.float32),
            jax.ShapeDtypeStruct((NP, 24), jnp.float32),
        ],
    )(xp, W, A, rxp8)


def _recip_body(d_ref, r_ref):
    r_ref[...] = 1.0 / (d_ref[...] + 1e-16)


def _tc_recip(den):
    return pl.pallas_call(
        _recip_body,
        out_shape=jax.ShapeDtypeStruct((NP, 16), jnp.float32),
    )(den)


_BN_SCALE = float((1.0 + 1e-5) ** -0.5)


def _mid_body(acc_ref, rd_ref, b_ref, g_ref, be_ref, w_ref, a_ref,
              h_ref, t_ref):
    blk = acc_ref.shape[0]
    r8 = rd_ref[...][:, :8]
    rexp = jnp.broadcast_to(r8[:, :, None], (blk, 8, 8)).reshape(blk, 64)
    s = acc_ref[...] * rexp + b_ref[...][None, :]
    s = s * (g_ref[...] * _BN_SCALE)[None, :] + be_ref[...][None, :]
    s = jnp.where(s > 0, s, jnp.exp(s) - 1.0)
    h = jnp.dot(s, w_ref[...], preferred_element_type=jnp.float32)
    h_ref[...] = h
    t_ref[...] = jnp.dot(h, a_ref[...], preferred_element_type=jnp.float32)


def _tc_mid(acc, rd, b, g, be, W, A):
    BLK = 2048
    return pl.pallas_call(
        _mid_body,
        grid=(NP // BLK,),
        in_specs=[
            pl.BlockSpec((BLK, 64), lambda i: (i, 0)),
            pl.BlockSpec((BLK, 16), lambda i: (i, 0)),
            pl.BlockSpec((64,), lambda i: (0,)),
            pl.BlockSpec((64,), lambda i: (0,)),
            pl.BlockSpec((64,), lambda i: (0,)),
            pl.BlockSpec((64, 64), lambda i: (0, 0)),
            pl.BlockSpec((64, 16), lambda i: (0, 0)),
        ],
        out_specs=[
            pl.BlockSpec((BLK, 64), lambda i: (i, 0)),
            pl.BlockSpec((BLK, 16), lambda i: (i, 0)),
        ],
        out_shape=[
            jax.ShapeDtypeStruct((NP, 64), jnp.float32),
            jax.ShapeDtypeStruct((NP, 16), jnp.float32),
        ],
    )(acc, rd, b, g, be, W, A)


def _tail_body(acc_ref, rd_ref, b_ref, g_ref, be_ref, batch_ref, wc_ref,
               bc_ref, out_ref, pool_ref, cnt_ref):
    i = pl.program_id(0)
    nb = pl.num_programs(0)

    @pl.when(i == 0)
    def _init():
        pool_ref[...] = jnp.zeros_like(pool_ref)
        cnt_ref[...] = jnp.zeros_like(cnt_ref)

    blk = acc_ref.shape[0]
    rexp = jnp.broadcast_to(rd_ref[...][:, :1], (blk, 64))
    s = acc_ref[...] * rexp + b_ref[...][None, :]
    s = s * (g_ref[...] * _BN_SCALE)[None, :] + be_ref[...][None, :]
    h = jnp.where(s > 0, s, jnp.exp(s) - 1.0)
    bvec = batch_ref[0, 0]
    onehot = (bvec[None, :] == lax.broadcasted_iota(
        jnp.int32, (NG, bvec.shape[0]), 0)).astype(jnp.float32)
    pool_ref[...] += jnp.dot(onehot, h, preferred_element_type=jnp.float32)
    cnt_ref[...] += jnp.sum(onehot, axis=1, keepdims=True)

    @pl.when(i == nb - 1)
    def _fin():
        gpool = pool_ref[...] / jnp.maximum(cnt_ref[...], 1.0)
        out_ref[...] = jnp.dot(
            gpool, wc_ref[...],
            preferred_element_type=jnp.float32) + bc_ref[...][None, :]


def _tc_tail(acc, rd, b, g, be, batch, Wc, bc):
    BLK = 2000
    return pl.pallas_call(
        _tail_body,
        grid=(N // BLK,),
        in_specs=[
            pl.BlockSpec((BLK, 64), lambda i: (i, 0)),
            pl.BlockSpec((BLK, 16), lambda i: (i, 0)),
            pl.BlockSpec((64,), lambda i: (0,)),
            pl.BlockSpec((64,), lambda i: (0,)),
            pl.BlockSpec((64,), lambda i: (0,)),
            pl.BlockSpec((1, 1, BLK), lambda i: (i, 0, 0)),
            pl.BlockSpec((HID, NC), lambda i: (0, 0)),
            pl.BlockSpec((NC,), lambda i: (0,)),
        ],
        out_specs=pl.BlockSpec((NG, NC), lambda i: (0, 0)),
        out_shape=jax.ShapeDtypeStruct((NG, NC), jnp.float32),
        scratch_shapes=[pltpu.VMEM((NG, HID), jnp.float32),
                        pltpu.VMEM((NG, 1), jnp.float32)],
    )(acc, rd, b, g, be, batch.reshape(N // BLK, 1, BLK), Wc, bc)


def kernel(x, edge_index, raw_x, batch, W1, att_src1, att_dst1, We1,
           att_edge1, b1, g1, be1, W2, att_src2, att_dst2, We2, att_edge2,
           b2, g2, be2, Wc, bc):
    src = edge_index[0]
    dst = edge_index[1]
    srcq, dstq, cnts = _pre_call()(src, dst)

    xp = jnp.pad(x, ((0, NP - N), (0, 0)))
    rxp8 = jnp.pad(raw_x, ((0, NP - N), (0, 5)))

    eye8 = jnp.eye(8, dtype=jnp.float32)
    As1 = (eye8[:, None, :] * att_src1[:, :, None]).reshape(64, 8)
    Ad1 = (eye8[:, None, :] * att_dst1[:, :, None]).reshape(64, 8)
    A1 = jnp.concatenate([As1, Ad1], axis=1)
    ce1 = (We1.reshape(HEADS, HD) * att_edge1).sum(-1)
    cev1 = jnp.repeat(ce1[:, None], 16, axis=1)

    A2 = jnp.zeros((64, 16), jnp.float32)
    A2 = A2.at[:, 0].set(att_src2[0]).at[:, 8].set(att_dst2[0])
    ce2 = (We2[0] * att_edge2[0]).sum()
    cev2 = jnp.full((1, 16), ce2, jnp.float32)

    h1, T1 = _tc_prep(xp, W1, A1, rxp8)
    accf1, denf1, ewq = _layer_call(HEADS, 24, True)(
        srcq, dstq, cnts, T1, h1, cev1)
    rden1 = _tc_recip(denf1.reshape(NP, 16))
    h2, T2 = _tc_mid(accf1.reshape(NP, 64), rden1, b1, g1, be1, W2, A2)
    accf2, denf2 = _layer_call(1, 16, False)(
        srcq, dstq, cnts, T2, h2, cev2, ewq)
    rden2 = _tc_recip(denf2.reshape(NP, 16))
    return _tc_tail(accf2.reshape(NP, 64), rden2, b2, g2, be2, batch, Wc, bc)


# speedup vs baseline: 1.1062x; 1.1062x over previous
import jax
import jax.numpy as jnp
from jax import lax
from jax.experimental import pallas as pl
from jax.experimental.pallas import tpu as pltpu
from jax.experimental.pallas import tpu_sc as plsc

N = 10000
E = 320000
IN = 128
HEADS = 8
HD = 8
HID = 64
NC = 10
NG = 16
SIGMA = 0.5

NP = 10240
BKT = 320
PAD_NODE = 10016
CAP = 12800
SCAN_C = 2000
C = 256

_SC_PARAMS = pltpu.CompilerParams(needs_layout_passes=False,
                                  use_tc_tiling_on_sc=False)


def _i16():
    return lax.broadcasted_iota(jnp.int32, (16,), 0)


def _f16(v):
    return jnp.full((16,), v, jnp.int32)


def _wid():
    return lax.axis_index("c") * 16 + lax.axis_index("s")


def _pre_call():
    def body(src_r, dst_r, srcq_o, dstq_o, cnts_o, sv, dv, sq, dq, cw):
        wid = _wid()
        lo = wid * BKT
        hi = lo + BKT

        def pf(i, c):
            o = pl.multiple_of(i * 16, 16)
            sq[pl.ds(o, 16)] = _f16(PAD_NODE)
            dq[pl.ds(o, 16)] = _f16(PAD_NODE)
            return c

        lax.fori_loop(0, (CAP + 16) // 16, pf, 0)

        def outer(ch, pos):
            pltpu.sync_copy(src_r.at[pl.ds(ch * SCAN_C, SCAN_C)], sv)
            pltpu.sync_copy(dst_r.at[pl.ds(ch * SCAN_C, SCAN_C)], dv)

            def inner(j, p):
                o = pl.multiple_of(j * 16, 16)
                s16 = sv[pl.ds(o, 16)]
                d16 = dv[pl.ds(o, 16)]
                m = (d16 >= lo) & (d16 < hi)
                plsc.store_compressed(sq.at[pl.ds(p, 16)], s16, mask=m)
                plsc.store_compressed(dq.at[pl.ds(p, 16)], d16, mask=m)
                pc = plsc.all_reduce_population_count(m)
                return p + pc[0]

            return lax.fori_loop(0, SCAN_C // 16, inner, pos)

        pos = lax.fori_loop(0, E // SCAN_C, outer, 0)
        pltpu.sync_copy(sq.at[pl.ds(0, CAP)], srcq_o.at[wid])
        pltpu.sync_copy(dq.at[pl.ds(0, CAP)], dstq_o.at[wid])
        cw[...] = jnp.zeros((16,), jnp.int32) + pos
        pltpu.sync_copy(cw, cnts_o.at[wid])

    return pl.kernel(
        body,
        out_type=(jax.ShapeDtypeStruct((32, CAP), jnp.int32),
                  jax.ShapeDtypeStruct((32, CAP), jnp.int32),
                  jax.ShapeDtypeStruct((32, 16), jnp.int32)),
        mesh=plsc.VectorSubcoreMesh(core_axis_name="c", subcore_axis_name="s"),
        scratch_types=(pltpu.VMEM((SCAN_C,), jnp.int32),
                       pltpu.VMEM((SCAN_C,), jnp.int32),
                       pltpu.VMEM((CAP + 16,), jnp.int32),
                       pltpu.VMEM((CAP + 16,), jnp.int32),
                       pltpu.VMEM((16,), jnp.int32)),
        compiler_params=_SC_PARAMS,
    )


def _layer_call(heads, tw, compute_ew):
    fan = 64 // heads
    out_type = [jax.ShapeDtypeStruct((NP * 64,), jnp.float32),
                jax.ShapeDtypeStruct((NP * 16,), jnp.float32)]
    if compute_ew:
        out_type.append(jax.ShapeDtypeStruct((32, CAP), jnp.float32))
    buf = lambda shape, dt: [pltpu.VMEM(shape, dt), pltpu.VMEM(shape, dt)]
    scratch = (
        buf((C,), jnp.int32) + buf((C,), jnp.int32)
        + buf((C, tw), jnp.float32) + buf((C, tw), jnp.float32)
        + buf((C, 64), jnp.float32) + buf((C,), jnp.float32)
        + [pltpu.VMEM((heads, 16), jnp.float32),
           pltpu.VMEM((16,), jnp.int32),
           pltpu.VMEM((BKT * 64,), jnp.float32),
           pltpu.VMEM((BKT * 16,), jnp.float32),
           pltpu.SemaphoreType.DMA]
    )

    def body(*refs):
        if compute_ew:
            (srcq_r, dstq_r, cnts_r, t_r, h_r, ce_r,
             acc_o, den_o, ewq_o,
             is0, is1, id0, id1, ts0, ts1, td0, td1, hv0, hv1, ew0, ew1,
             cevv, cw, accT, denT, sem) = refs
            ewq_r = None
        else:
            (srcq_r, dstq_r, cnts_r, t_r, h_r, ce_r, ewq_r,
             acc_o, den_o,
             is0, is1, id0, id1, ts0, ts1, td0, td1, hv0, hv1, ew0, ew1,
             cevv, cw, accT, denT, sem) = refs
            ewq_o = None
        bufs = ((is0, id0, ts0, td0, hv0, ew0), (is1, id1, ts1, td1, hv1, ew1))
        wid = _wid()
        lo = wid * BKT

        def z64(i, c):
            o = pl.multiple_of(i * 16, 16)
            accT[pl.ds(o, 16)] = jnp.zeros((16,), jnp.float32)
            return c

        def z16(i, c):
            o = pl.multiple_of(i * 16, 16)
            denT[pl.ds(o, 16)] = jnp.zeros((16,), jnp.float32)
            return c

        lax.fori_loop(0, BKT * 4, z64, 0)
        lax.fori_loop(0, BKT, z16, 0)
        pltpu.sync_copy(ce_r, cevv)
        pltpu.sync_copy(cnts_r.at[wid], cw)
        cnt = jnp.max(cw[...])
        nch = (cnt + (C - 1)) // C
        ce_b = [cevv[h] for h in range(heads)]
        i16 = _i16()

        def fire(ci, B):
            bis, bid, bts, btd, bhv, bew = B
            off = ci * C
            pltpu.sync_copy(srcq_r.at[wid, pl.ds(off, C)], bis)
            pltpu.sync_copy(dstq_r.at[wid, pl.ds(off, C)], bid)
            if not compute_ew:
                pltpu.sync_copy(ewq_r.at[wid, pl.ds(off, C)], bew)
            for b in range(C // 128):
                sl = pl.ds(b * 128, 128)
                pltpu.async_copy(t_r.at[bis.at[sl]], bts.at[sl], sem)
                pltpu.async_copy(t_r.at[bid.at[sl]], btd.at[sl], sem)
                pltpu.async_copy(h_r.at[bis.at[sl]], bhv.at[sl], sem)

        def wait_for(B):
            bis, bid, bts, btd, bhv, bew = B
            for b in range(C // 128):
                sl = pl.ds(b * 128, 128)
                pltpu.make_async_copy(t_r.at[bis.at[sl]], bts.at[sl], sem).wait()
                pltpu.make_async_copy(t_r.at[bid.at[sl]], btd.at[sl], sem).wait()
                pltpu.make_async_copy(h_r.at[bis.at[sl]], bhv.at[sl], sem).wait()

        def compute(ci, B):
            bis, bid, bts, btd, bhv, bew = B
            off = ci * C

            def jb(j, c2):
                o = pl.multiple_of(j * 16, 16)
                rows = j * 16 + i16
                msk = (off + rows) < cnt
                lr = bid[pl.ds(o, 16)] - lo
                if compute_ew:
                    dacc = None
                    for k in range(3):
                        kf = _f16(16 + k)
                        df = (plsc.load_gather(bts, [rows, kf])
                              - plsc.load_gather(btd, [rows, kf]))
                        dacc = df * df if dacc is None else dacc + df * df
                    ew16 = jnp.exp(dacc * (-1.0 / (2.0 * SIGMA * SIGMA)))
                    bew[pl.ds(o, 16)] = ew16
                else:
                    ew16 = bew[pl.ds(o, 16)]
                for h in range(heads):
                    a = (plsc.load_gather(bts, [rows, _f16(h)])
                         + plsc.load_gather(btd, [rows, _f16(h + 8)])
                         + ce_b[h] * ew16)
                    a = jnp.maximum(a, a * 0.2)
                    ex = jnp.exp(a)
                    plsc.addupdate_scatter(denT, [lr * 16 + h], ex, mask=msk)
                    for dd in range(fan):
                        col = h * fan + dd
                        v = plsc.load_gather(bhv, [rows, _f16(col)]) * ex
                        plsc.addupdate_scatter(accT, [lr * 64 + col], v,
                                               mask=msk)
                return c2

            lax.fori_loop(0, C // 16, jb, 0)
            if compute_ew:
                pltpu.sync_copy(bew, ewq_o.at[wid, pl.ds(off, C)])

        @pl.when(nch > 0)
        def _prologue():
            fire(0, bufs[0])

        def pair(i, c):
            ci_a = 2 * i
            ci_b = ci_a + 1
            wait_for(bufs[0])

            @pl.when(ci_b < nch)
            def _f1():
                fire(ci_b, bufs[1])

            compute(ci_a, bufs[0])

            @pl.when(ci_b < nch)
            def _p2():
                wait_for(bufs[1])

                @pl.when(ci_b + 1 < nch)
                def _f2():
                    fire(ci_b + 1, bufs[0])

                compute(ci_b, bufs[1])

            return c

        lax.fori_loop(0, (nch + 1) // 2, pair, 0)
        pltpu.sync_copy(accT, acc_o.at[pl.ds(lo * 64, BKT * 64)])
        pltpu.sync_copy(denT, den_o.at[pl.ds(lo * 16, BKT * 16)])

    return pl.kernel(
        body,
        out_type=tuple(out_type),
        mesh=plsc.VectorSubcoreMesh(core_axis_name="c", subcore_axis_name="s"),
        scratch_types=tuple(scratch),
        compiler_params=_SC_PARAMS,
    )


def _prep_body(x_ref, w_ref, a_ref, rx_ref, h_ref, t_ref):
    h = jnp.dot(x_ref[...], w_ref[...], preferred_element_type=jnp.float32)
    h_ref[...] = h
    al = jnp.dot(h, a_ref[...], preferred_element_type=jnp.float32)
    t_ref[...] = jnp.concatenate([al, rx_ref[...]], axis=1)


def _tc_prep(xp, W, A, rxp8):
    BLK = 2048
    return pl.pallas_call(
        _prep_body,
        grid=(NP // BLK,),
        in_specs=[
            pl.BlockSpec((BLK, IN), lambda i: (i, 0)),
            pl.BlockSpec((IN, 64), lambda i: (0, 0)),
            pl.BlockSpec((64, 16), lambda i: (0, 0)),
            pl.BlockSpec((BLK, 8), lambda i: (i, 0)),
        ],
        out_specs=[
            pl.BlockSpec((BLK, 64), lambda i: (i, 0)),
            pl.BlockSpec((BLK, 24), lambda i: (i, 0)),
        ],
        out_shape=[
            jax.ShapeDtypeStruct((NP, 64), jnp.float32),
            jax.ShapeDtypeStruct((NP, 24), jnp.float32),
        ],
    )(xp, W, A, rxp8)


def _recip_body(d_ref, r_ref):
    r_ref[...] = 1.0 / (d_ref[...] + 1e-16)


def _tc_recip(den):
    return pl.pallas_call(
        _recip_body,
        out_shape=jax.ShapeDtypeStruct((NP, 16), jnp.float32),
    )(den)


_BN_SCALE = float((1.0 + 1e-5) ** -0.5)


def _mid_body(acc_ref, rd_ref, b_ref, g_ref, be_ref, w_ref, a_ref,
              h_ref, t_ref):
    blk = acc_ref.shape[0]
    r8 = rd_ref[...][:, :8]
    rexp = jnp.broadcast_to(r8[:, :, None], (blk, 8, 8)).reshape(blk, 64)
    s = acc_ref[...] * rexp + b_ref[...][None, :]
    s = s * (g_ref[...] * _BN_SCALE)[None, :] + be_ref[...][None, :]
    s = jnp.where(s > 0, s, jnp.exp(s) - 1.0)
    h = jnp.dot(s, w_ref[...], preferred_element_type=jnp.float32)
    h_ref[...] = h
    t_ref[...] = jnp.dot(h, a_ref[...], preferred_element_type=jnp.float32)


def _tc_mid(acc, rd, b, g, be, W, A):
    BLK = 2048
    return pl.pallas_call(
        _mid_body,
        grid=(NP // BLK,),
        in_specs=[
            pl.BlockSpec((BLK, 64), lambda i: (i, 0)),
            pl.BlockSpec((BLK, 16), lambda i: (i, 0)),
            pl.BlockSpec((64,), lambda i: (0,)),
            pl.BlockSpec((64,), lambda i: (0,)),
            pl.BlockSpec((64,), lambda i: (0,)),
            pl.BlockSpec((64, 64), lambda i: (0, 0)),
            pl.BlockSpec((64, 16), lambda i: (0, 0)),
        ],
        out_specs=[
            pl.BlockSpec((BLK, 64), lambda i: (i, 0)),
            pl.BlockSpec((BLK, 16), lambda i: (i, 0)),
        ],
        out_shape=[
            jax.ShapeDtypeStruct((NP, 64), jnp.float32),
            jax.ShapeDtypeStruct((NP, 16), jnp.float32),
        ],
    )(acc, rd, b, g, be, W, A)


def _tail_body(acc_ref, rd_ref, b_ref, g_ref, be_ref, batch_ref, wc_ref,
               bc_ref, out_ref, pool_ref, cnt_ref):
    i = pl.program_id(0)
    nb = pl.num_programs(0)

    @pl.when(i == 0)
    def _init():
        pool_ref[...] = jnp.zeros_like(pool_ref)
        cnt_ref[...] = jnp.zeros_like(cnt_ref)

    blk = acc_ref.shape[0]
    rexp = jnp.broadcast_to(rd_ref[...][:, :1], (blk, 64))
    s = acc_ref[...] * rexp + b_ref[...][None, :]
    s = s * (g_ref[...] * _BN_SCALE)[None, :] + be_ref[...][None, :]
    h = jnp.where(s > 0, s, jnp.exp(s) - 1.0)
    bvec = batch_ref[0, 0]
    onehot = (bvec[None, :] == lax.broadcasted_iota(
        jnp.int32, (NG, bvec.shape[0]), 0)).astype(jnp.float32)
    pool_ref[...] += jnp.dot(onehot, h, preferred_element_type=jnp.float32)
    cnt_ref[...] += jnp.sum(onehot, axis=1, keepdims=True)

    @pl.when(i == nb - 1)
    def _fin():
        gpool = pool_ref[...] / jnp.maximum(cnt_ref[...], 1.0)
        out_ref[...] = jnp.dot(
            gpool, wc_ref[...],
            preferred_element_type=jnp.float32) + bc_ref[...][None, :]


def _tc_tail(acc, rd, b, g, be, batch, Wc, bc):
    BLK = 2000
    return pl.pallas_call(
        _tail_body,
        grid=(N // BLK,),
        in_specs=[
            pl.BlockSpec((BLK, 64), lambda i: (i, 0)),
            pl.BlockSpec((BLK, 16), lambda i: (i, 0)),
            pl.BlockSpec((64,), lambda i: (0,)),
            pl.BlockSpec((64,), lambda i: (0,)),
            pl.BlockSpec((64,), lambda i: (0,)),
            pl.BlockSpec((1, 1, BLK), lambda i: (i, 0, 0)),
            pl.BlockSpec((HID, NC), lambda i: (0, 0)),
            pl.BlockSpec((NC,), lambda i: (0,)),
        ],
        out_specs=pl.BlockSpec((NG, NC), lambda i: (0, 0)),
        out_shape=jax.ShapeDtypeStruct((NG, NC), jnp.float32),
        scratch_shapes=[pltpu.VMEM((NG, HID), jnp.float32),
                        pltpu.VMEM((NG, 1), jnp.float32)],
    )(acc, rd, b, g, be, batch.reshape(N // BLK, 1, BLK), Wc, bc)


def kernel(x, edge_index, raw_x, batch, W1, att_src1, att_dst1, We1,
           att_edge1, b1, g1, be1, W2, att_src2, att_dst2, We2, att_edge2,
           b2, g2, be2, Wc, bc):
    src = edge_index[0]
    dst = edge_index[1]
    srcq, dstq, cnts = _pre_call()(src, dst)

    xp = jnp.pad(x, ((0, NP - N), (0, 0)))
    rxp8 = jnp.pad(raw_x, ((0, NP - N), (0, 5)))

    eye8 = jnp.eye(8, dtype=jnp.float32)
    As1 = (eye8[:, None, :] * att_src1[:, :, None]).reshape(64, 8)
    Ad1 = (eye8[:, None, :] * att_dst1[:, :, None]).reshape(64, 8)
    A1 = jnp.concatenate([As1, Ad1], axis=1)
    ce1 = (We1.reshape(HEADS, HD) * att_edge1).sum(-1)
    cev1 = jnp.repeat(ce1[:, None], 16, axis=1)

    A2 = jnp.zeros((64, 16), jnp.float32)
    A2 = A2.at[:, 0].set(att_src2[0]).at[:, 8].set(att_dst2[0])
    ce2 = (We2[0] * att_edge2[0]).sum()
    cev2 = jnp.full((1, 16), ce2, jnp.float32)

    h1, T1 = _tc_prep(xp, W1, A1, rxp8)
    accf1, denf1, ewq = _layer_call(HEADS, 24, True)(
        srcq, dstq, cnts, T1, h1, cev1)
    rden1 = _tc_recip(denf1.reshape(NP, 16))
    h2, T2 = _tc_mid(accf1.reshape(NP, 64), rden1, b1, g1, be1, W2, A2)
    accf2, denf2 = _layer_call(1, 16, False)(
        srcq, dstq, cnts, T2, h2, cev2, ewq)
    rden2 = _tc_recip(denf2.reshape(NP, 16))
    return _tc_tail(accf2.reshape(NP, 64), rden2, b2, g2, be2, batch, Wc, bc)


# speedup vs baseline: 1.5323x; 1.3852x over previous
import jax
import jax.numpy as jnp
from jax import lax
from jax.experimental import pallas as pl
from jax.experimental.pallas import tpu as pltpu
from jax.experimental.pallas import tpu_sc as plsc

N = 10000
E = 320000
IN = 128
HEADS = 8
HD = 8
HID = 64
NC = 10
NG = 16
SIGMA = 0.5

NP = 10240
BKT = 320
PAD_NODE = 10016
CAP = 12800
SCAN_C = 8000
C = 256

_SC_PARAMS = pltpu.CompilerParams(needs_layout_passes=False,
                                  use_tc_tiling_on_sc=False)


def _i16():
    return lax.broadcasted_iota(jnp.int32, (16,), 0)


def _f16(v):
    return jnp.full((16,), v, jnp.int32)


def _wid():
    return lax.axis_index("c") * 16 + lax.axis_index("s")


def _pre_call():
    def body(src_r, dst_r, srcq_o, dstq_o, cnts_o, sv, dv, sq, dq, cw):
        wid = _wid()
        lo = wid * BKT
        hi = lo + BKT

        def pf(i, c):
            o = pl.multiple_of(i * 16, 16)
            sq[pl.ds(o, 16)] = _f16(PAD_NODE)
            dq[pl.ds(o, 16)] = _f16(PAD_NODE)
            return c

        lax.fori_loop(0, (CAP + 16) // 16, pf, 0)

        def outer(ch, pos):
            pltpu.sync_copy(src_r.at[pl.ds(ch * SCAN_C, SCAN_C)], sv)
            pltpu.sync_copy(dst_r.at[pl.ds(ch * SCAN_C, SCAN_C)], dv)

            def inner(j, p):
                o = pl.multiple_of(j * 16, 16)
                s16 = sv[pl.ds(o, 16)]
                d16 = dv[pl.ds(o, 16)]
                m = (d16 >= lo) & (d16 < hi)
                plsc.store_compressed(sq.at[pl.ds(p, 16)], s16, mask=m)
                plsc.store_compressed(dq.at[pl.ds(p, 16)], d16, mask=m)
                pc = plsc.all_reduce_population_count(m)
                return p + pc[0]

            return lax.fori_loop(0, SCAN_C // 16, inner, pos)

        pos = lax.fori_loop(0, E // SCAN_C, outer, 0)
        pltpu.sync_copy(sq.at[pl.ds(0, CAP)], srcq_o.at[wid])
        pltpu.sync_copy(dq.at[pl.ds(0, CAP)], dstq_o.at[wid])
        cw[...] = jnp.zeros((16,), jnp.int32) + pos
        pltpu.sync_copy(cw, cnts_o.at[wid])

    return pl.kernel(
        body,
        out_type=(jax.ShapeDtypeStruct((32, CAP), jnp.int32),
                  jax.ShapeDtypeStruct((32, CAP), jnp.int32),
                  jax.ShapeDtypeStruct((32, 16), jnp.int32)),
        mesh=plsc.VectorSubcoreMesh(core_axis_name="c", subcore_axis_name="s"),
        scratch_types=(pltpu.VMEM((SCAN_C,), jnp.int32),
                       pltpu.VMEM((SCAN_C,), jnp.int32),
                       pltpu.VMEM((CAP + 16,), jnp.int32),
                       pltpu.VMEM((CAP + 16,), jnp.int32),
                       pltpu.VMEM((16,), jnp.int32)),
        compiler_params=_SC_PARAMS,
    )


def _layer_call(heads, tw, compute_ew):
    fan = 64 // heads
    out_type = [jax.ShapeDtypeStruct((NP * 65,), jnp.float32),
                jax.ShapeDtypeStruct((NP * 17,), jnp.float32)]
    if compute_ew:
        out_type.append(jax.ShapeDtypeStruct((32, CAP), jnp.float32))
    buf = lambda shape, dt: [pltpu.VMEM(shape, dt), pltpu.VMEM(shape, dt)]
    scratch = (
        buf((C,), jnp.int32) + buf((C,), jnp.int32)
        + buf((C, tw), jnp.float32) + buf((C, tw), jnp.float32)
        + buf((C, 64), jnp.float32) + buf((C,), jnp.float32)
        + [pltpu.VMEM((heads, 16), jnp.float32),
           pltpu.VMEM((16,), jnp.int32),
           pltpu.VMEM((BKT * 65,), jnp.float32),
           pltpu.VMEM((BKT * 17,), jnp.float32),
           pltpu.SemaphoreType.DMA]
    )

    def body(*refs):
        if compute_ew:
            (srcq_r, dstq_r, cnts_r, t_r, h_r, ce_r,
             acc_o, den_o, ewq_o,
             is0, is1, id0, id1, ts0, ts1, td0, td1, hv0, hv1, ew0, ew1,
             cevv, cw, accT, denT, sem) = refs
            ewq_r = None
        else:
            (srcq_r, dstq_r, cnts_r, t_r, h_r, ce_r, ewq_r,
             acc_o, den_o,
             is0, is1, id0, id1, ts0, ts1, td0, td1, hv0, hv1, ew0, ew1,
             cevv, cw, accT, denT, sem) = refs
            ewq_o = None
        bufs = ((is0, id0, ts0, td0, hv0, ew0), (is1, id1, ts1, td1, hv1, ew1))
        wid = _wid()
        lo = wid * BKT

        def z65(i, c):
            o = pl.multiple_of(i * 16, 16)
            accT[pl.ds(o, 16)] = jnp.zeros((16,), jnp.float32)
            return c

        def z17(i, c):
            o = pl.multiple_of(i * 16, 16)
            denT[pl.ds(o, 16)] = jnp.zeros((16,), jnp.float32)
            return c

        lax.fori_loop(0, BKT * 65 // 16, z65, 0)
        lax.fori_loop(0, BKT * 17 // 16, z17, 0)
        pltpu.sync_copy(ce_r, cevv)
        pltpu.sync_copy(cnts_r.at[wid], cw)
        cnt = jnp.max(cw[...])
        nch = (cnt + (C - 1)) // C
        ce_b = [cevv[h] for h in range(heads)]
        i16 = _i16()

        def fire(ci, B):
            bis, bid, bts, btd, bhv, bew = B
            off = ci * C
            pltpu.sync_copy(srcq_r.at[wid, pl.ds(off, C)], bis)
            pltpu.sync_copy(dstq_r.at[wid, pl.ds(off, C)], bid)
            if not compute_ew:
                pltpu.sync_copy(ewq_r.at[wid, pl.ds(off, C)], bew)
            for b in range(C // 128):
                sl = pl.ds(b * 128, 128)
                pltpu.async_copy(t_r.at[bis.at[sl]], bts.at[sl], sem)
                pltpu.async_copy(t_r.at[bid.at[sl]], btd.at[sl], sem)
                pltpu.async_copy(h_r.at[bis.at[sl]], bhv.at[sl], sem)

        def wait_for(B):
            bis, bid, bts, btd, bhv, bew = B
            for b in range(C // 128):
                sl = pl.ds(b * 128, 128)
                pltpu.make_async_copy(t_r.at[bis.at[sl]], bts.at[sl], sem).wait()
                pltpu.make_async_copy(t_r.at[bid.at[sl]], btd.at[sl], sem).wait()
                pltpu.make_async_copy(h_r.at[bis.at[sl]], bhv.at[sl], sem).wait()

        def compute(ci, B):
            bis, bid, bts, btd, bhv, bew = B
            off = ci * C

            def jb(j, c2):
                o = pl.multiple_of(j * 16, 16)
                rows = j * 16 + i16
                msk = (off + rows) < cnt
                lr = bid[pl.ds(o, 16)] - lo
                if compute_ew:
                    dacc = None
                    for k in range(3):
                        kf = _f16(16 + k)
                        df = (plsc.load_gather(bts, [rows, kf])
                              - plsc.load_gather(btd, [rows, kf]))
                        dacc = df * df if dacc is None else dacc + df * df
                    ew16 = jnp.exp(dacc * (-1.0 / (2.0 * SIGMA * SIGMA)))
                    bew[pl.ds(o, 16)] = ew16
                else:
                    ew16 = bew[pl.ds(o, 16)]
                for h in range(heads):
                    a = (plsc.load_gather(bts, [rows, _f16(h)])
                         + plsc.load_gather(btd, [rows, _f16(h + 8)])
                         + ce_b[h] * ew16)
                    a = jnp.maximum(a, a * 0.2)
                    ex = jnp.exp(a)
                    plsc.addupdate_scatter(denT, [lr * 17 + h], ex, mask=msk)
                    for dd in range(fan):
                        col = h * fan + dd
                        v = plsc.load_gather(bhv, [rows, _f16(col)]) * ex
                        plsc.addupdate_scatter(accT, [lr * 65 + col], v,
                                               mask=msk)
                return c2

            lax.fori_loop(0, C // 16, jb, 0)
            if compute_ew:
                pltpu.sync_copy(bew, ewq_o.at[wid, pl.ds(off, C)])

        @pl.when(nch > 0)
        def _prologue():
            fire(0, bufs[0])

        def pair(i, c):
            ci_a = 2 * i
            ci_b = ci_a + 1
            wait_for(bufs[0])

            @pl.when(ci_b < nch)
            def _f1():
                fire(ci_b, bufs[1])

            compute(ci_a, bufs[0])

            @pl.when(ci_b < nch)
            def _p2():
                wait_for(bufs[1])

                @pl.when(ci_b + 1 < nch)
                def _f2():
                    fire(ci_b + 1, bufs[0])

                compute(ci_b, bufs[1])

            return c

        lax.fori_loop(0, (nch + 1) // 2, pair, 0)
        pltpu.sync_copy(accT, acc_o.at[pl.ds(lo * 65, BKT * 65)])
        pltpu.sync_copy(denT, den_o.at[pl.ds(lo * 17, BKT * 17)])

    return pl.kernel(
        body,
        out_type=tuple(out_type),
        mesh=plsc.VectorSubcoreMesh(core_axis_name="c", subcore_axis_name="s"),
        scratch_types=tuple(scratch),
        compiler_params=_SC_PARAMS,
    )


def _prep_body(x_ref, w_ref, a_ref, rx_ref, h_ref, t_ref):
    h = jnp.dot(x_ref[...], w_ref[...], preferred_element_type=jnp.float32)
    h_ref[...] = h
    al = jnp.dot(h, a_ref[...], preferred_element_type=jnp.float32)
    t_ref[...] = jnp.concatenate([al, rx_ref[...]], axis=1)


def _tc_prep(xp, W, A, rxp8):
    BLK = 2048
    return pl.pallas_call(
        _prep_body,
        grid=(NP // BLK,),
        in_specs=[
            pl.BlockSpec((BLK, IN), lambda i: (i, 0)),
            pl.BlockSpec((IN, 64), lambda i: (0, 0)),
            pl.BlockSpec((64, 16), lambda i: (0, 0)),
            pl.BlockSpec((BLK, 8), lambda i: (i, 0)),
        ],
        out_specs=[
            pl.BlockSpec((BLK, 64), lambda i: (i, 0)),
            pl.BlockSpec((BLK, 24), lambda i: (i, 0)),
        ],
        out_shape=[
            jax.ShapeDtypeStruct((NP, 64), jnp.float32),
            jax.ShapeDtypeStruct((NP, 24), jnp.float32),
        ],
    )(xp, W, A, rxp8)


def _recip_body(d_ref, r_ref):
    r_ref[...] = 1.0 / (d_ref[...] + 1e-16)


def _tc_recip(den):
    return pl.pallas_call(
        _recip_body,
        out_shape=jax.ShapeDtypeStruct((NP, 17), jnp.float32),
    )(den)


_BN_SCALE = float((1.0 + 1e-5) ** -0.5)


def _mid_body(acc_ref, rd_ref, b_ref, g_ref, be_ref, w_ref, a_ref,
              h_ref, t_ref):
    blk = acc_ref.shape[0]
    r8 = rd_ref[...][:, :8]
    rexp = jnp.broadcast_to(r8[:, :, None], (blk, 8, 8)).reshape(blk, 64)
    s = acc_ref[...][:, :64] * rexp + b_ref[...][None, :]
    s = s * (g_ref[...] * _BN_SCALE)[None, :] + be_ref[...][None, :]
    s = jnp.where(s > 0, s, jnp.exp(s) - 1.0)
    h = jnp.dot(s, w_ref[...], preferred_element_type=jnp.float32)
    h_ref[...] = h
    t_ref[...] = jnp.dot(h, a_ref[...], preferred_element_type=jnp.float32)


def _tc_mid(acc, rd, b, g, be, W, A):
    BLK = 2048
    return pl.pallas_call(
        _mid_body,
        grid=(NP // BLK,),
        in_specs=[
            pl.BlockSpec((BLK, 65), lambda i: (i, 0)),
            pl.BlockSpec((BLK, 17), lambda i: (i, 0)),
            pl.BlockSpec((64,), lambda i: (0,)),
            pl.BlockSpec((64,), lambda i: (0,)),
            pl.BlockSpec((64,), lambda i: (0,)),
            pl.BlockSpec((64, 64), lambda i: (0, 0)),
            pl.BlockSpec((64, 16), lambda i: (0, 0)),
        ],
        out_specs=[
            pl.BlockSpec((BLK, 64), lambda i: (i, 0)),
            pl.BlockSpec((BLK, 16), lambda i: (i, 0)),
        ],
        out_shape=[
            jax.ShapeDtypeStruct((NP, 64), jnp.float32),
            jax.ShapeDtypeStruct((NP, 16), jnp.float32),
        ],
    )(acc, rd, b, g, be, W, A)


def _tail_body(acc_ref, rd_ref, b_ref, g_ref, be_ref, batch_ref, wc_ref,
               bc_ref, out_ref, pool_ref, cnt_ref):
    i = pl.program_id(0)
    nb = pl.num_programs(0)

    @pl.when(i == 0)
    def _init():
        pool_ref[...] = jnp.zeros_like(pool_ref)
        cnt_ref[...] = jnp.zeros_like(cnt_ref)

    blk = acc_ref.shape[0]
    rexp = jnp.broadcast_to(rd_ref[...][:, :1], (blk, 64))
    s = acc_ref[...][:, :64] * rexp + b_ref[...][None, :]
    s = s * (g_ref[...] * _BN_SCALE)[None, :] + be_ref[...][None, :]
    h = jnp.where(s > 0, s, jnp.exp(s) - 1.0)
    bvec = batch_ref[0, 0]
    onehot = (bvec[None, :] == lax.broadcasted_iota(
        jnp.int32, (NG, bvec.shape[0]), 0)).astype(jnp.float32)
    pool_ref[...] += jnp.dot(onehot, h, preferred_element_type=jnp.float32)
    cnt_ref[...] += jnp.sum(onehot, axis=1, keepdims=True)

    @pl.when(i == nb - 1)
    def _fin():
        gpool = pool_ref[...] / jnp.maximum(cnt_ref[...], 1.0)
        out_ref[...] = jnp.dot(
            gpool, wc_ref[...],
            preferred_element_type=jnp.float32) + bc_ref[...][None, :]


def _tc_tail(acc, rd, b, g, be, batch, Wc, bc):
    BLK = 2000
    return pl.pallas_call(
        _tail_body,
        grid=(N // BLK,),
        in_specs=[
            pl.BlockSpec((BLK, 65), lambda i: (i, 0)),
            pl.BlockSpec((BLK, 17), lambda i: (i, 0)),
            pl.BlockSpec((64,), lambda i: (0,)),
            pl.BlockSpec((64,), lambda i: (0,)),
            pl.BlockSpec((64,), lambda i: (0,)),
            pl.BlockSpec((1, 1, BLK), lambda i: (i, 0, 0)),
            pl.BlockSpec((HID, NC), lambda i: (0, 0)),
            pl.BlockSpec((NC,), lambda i: (0,)),
        ],
        out_specs=pl.BlockSpec((NG, NC), lambda i: (0, 0)),
        out_shape=jax.ShapeDtypeStruct((NG, NC), jnp.float32),
        scratch_shapes=[pltpu.VMEM((NG, HID), jnp.float32),
                        pltpu.VMEM((NG, 1), jnp.float32)],
    )(acc, rd, b, g, be, batch.reshape(N // BLK, 1, BLK), Wc, bc)


def kernel(x, edge_index, raw_x, batch, W1, att_src1, att_dst1, We1,
           att_edge1, b1, g1, be1, W2, att_src2, att_dst2, We2, att_edge2,
           b2, g2, be2, Wc, bc):
    src = edge_index[0]
    dst = edge_index[1]
    srcq, dstq, cnts = _pre_call()(src, dst)

    xp = jnp.pad(x, ((0, NP - N), (0, 0)))
    rxp8 = jnp.pad(raw_x, ((0, NP - N), (0, 5)))

    eye8 = jnp.eye(8, dtype=jnp.float32)
    As1 = (eye8[:, None, :] * att_src1[:, :, None]).reshape(64, 8)
    Ad1 = (eye8[:, None, :] * att_dst1[:, :, None]).reshape(64, 8)
    A1 = jnp.concatenate([As1, Ad1], axis=1)
    ce1 = (We1.reshape(HEADS, HD) * att_edge1).sum(-1)
    cev1 = jnp.repeat(ce1[:, None], 16, axis=1)

    A2 = jnp.zeros((64, 16), jnp.float32)
    A2 = A2.at[:, 0].set(att_src2[0]).at[:, 8].set(att_dst2[0])
    ce2 = (We2[0] * att_edge2[0]).sum()
    cev2 = jnp.full((1, 16), ce2, jnp.float32)

    h1, T1 = _tc_prep(xp, W1, A1, rxp8)
    accf1, denf1, ewq = _layer_call(HEADS, 24, True)(
        srcq, dstq, cnts, T1, h1, cev1)
    rden1 = _tc_recip(denf1.reshape(NP, 17))
    h2, T2 = _tc_mid(accf1.reshape(NP, 65), rden1, b1, g1, be1, W2, A2)
    accf2, denf2 = _layer_call(1, 16, False)(
        srcq, dstq, cnts, T2, h2, cev2, ewq)
    rden2 = _tc_recip(denf2.reshape(NP, 17))
    return _tc_tail(accf2.reshape(NP, 65), rden2, b2, g2, be2, batch, Wc, bc)


# speedup vs baseline: 2.3386x; 1.5261x over previous
import jax
import jax.numpy as jnp
from jax import lax
from jax.experimental import pallas as pl
from jax.experimental.pallas import tpu as pltpu
from jax.experimental.pallas import tpu_sc as plsc

N = 10000
E = 320000
IN = 128
HEADS = 8
HD = 8
HID = 64
NC = 10
NG = 16
SIGMA = 0.5

NP = 10240
BKT = 320
PAD_NODE = 10016
CAP = 12800
SCAN_C = 8000
C = 256

LOG2F = {8: 3, 64: 6}

_SC_PARAMS = pltpu.CompilerParams(needs_layout_passes=False,
                                  use_tc_tiling_on_sc=False)


def _i16():
    return lax.broadcasted_iota(jnp.int32, (16,), 0)


def _f16(v):
    return jnp.full((16,), v, jnp.int32)


def _wid():
    return lax.axis_index("c") * 16 + lax.axis_index("s")


def _pre_call():
    def body(src_r, dst_r, srcq_o, dstq_o, cnts_o, sv, dv, sq, dq, cw):
        wid = _wid()
        lo = wid * BKT
        hi = lo + BKT

        def pf(i, c):
            o = pl.multiple_of(i * 16, 16)
            sq[pl.ds(o, 16)] = _f16(PAD_NODE)
            dq[pl.ds(o, 16)] = _f16(PAD_NODE)
            return c

        lax.fori_loop(0, (CAP + 16) // 16, pf, 0)

        def outer(ch, pos):
            pltpu.sync_copy(src_r.at[pl.ds(ch * SCAN_C, SCAN_C)], sv)
            pltpu.sync_copy(dst_r.at[pl.ds(ch * SCAN_C, SCAN_C)], dv)

            def inner(j, p):
                o = pl.multiple_of(j * 16, 16)
                s16 = sv[pl.ds(o, 16)]
                d16 = dv[pl.ds(o, 16)]
                m = (d16 >= lo) & (d16 < hi)
                plsc.store_compressed(sq.at[pl.ds(p, 16)], s16, mask=m)
                plsc.store_compressed(dq.at[pl.ds(p, 16)], d16, mask=m)
                pc = plsc.all_reduce_population_count(m)
                return p + pc[0]

            return lax.fori_loop(0, SCAN_C // 16, inner, pos)

        pos = lax.fori_loop(0, E // SCAN_C, outer, 0)
        pltpu.sync_copy(sq.at[pl.ds(0, CAP)], srcq_o.at[wid])
        pltpu.sync_copy(dq.at[pl.ds(0, CAP)], dstq_o.at[wid])
        cw[...] = jnp.zeros((16,), jnp.int32) + pos
        pltpu.sync_copy(cw, cnts_o.at[wid])

    return pl.kernel(
        body,
        out_type=(jax.ShapeDtypeStruct((32, CAP), jnp.int32),
                  jax.ShapeDtypeStruct((32, CAP), jnp.int32),
                  jax.ShapeDtypeStruct((32, 16), jnp.int32)),
        mesh=plsc.VectorSubcoreMesh(core_axis_name="c", subcore_axis_name="s"),
        scratch_types=(pltpu.VMEM((SCAN_C,), jnp.int32),
                       pltpu.VMEM((SCAN_C,), jnp.int32),
                       pltpu.VMEM((CAP + 16,), jnp.int32),
                       pltpu.VMEM((CAP + 16,), jnp.int32),
                       pltpu.VMEM((16,), jnp.int32)),
        compiler_params=_SC_PARAMS,
    )


def _layer_call(heads, tw, compute_ew):
    fan = 64 // heads
    out_type = [jax.ShapeDtypeStruct((NP * 65,), jnp.float32),
                jax.ShapeDtypeStruct((NP * 17,), jnp.float32)]
    if compute_ew:
        out_type.append(jax.ShapeDtypeStruct((32, CAP), jnp.float32))
    buf = lambda shape, dt: [pltpu.VMEM(shape, dt), pltpu.VMEM(shape, dt)]
    scratch = (
        buf((C,), jnp.int32) + buf((C,), jnp.int32)
        + buf((C, tw), jnp.float32) + buf((C, tw), jnp.float32)
        + buf((C, 64), jnp.float32) + buf((C,), jnp.float32)
        + [pltpu.VMEM((heads, 16), jnp.float32),
           pltpu.VMEM((16,), jnp.int32),
           pltpu.VMEM((BKT * 65,), jnp.float32),
           pltpu.VMEM((BKT * 17,), jnp.float32),
           pltpu.VMEM((C * 17,), jnp.float32),
           pltpu.SemaphoreType.DMA]
    )

    def body(*refs):
        if compute_ew:
            (srcq_r, dstq_r, cnts_r, t_r, h_r, ce_r,
             acc_o, den_o, ewq_o,
             is0, is1, id0, id1, ts0, ts1, td0, td1, hv0, hv1, ew0, ew1,
             cevv, cw, accT, denT, stageF, sem) = refs
            ewq_r = None
        else:
            (srcq_r, dstq_r, cnts_r, t_r, h_r, ce_r, ewq_r,
             acc_o, den_o,
             is0, is1, id0, id1, ts0, ts1, td0, td1, hv0, hv1, ew0, ew1,
             cevv, cw, accT, denT, stageF, sem) = refs
            ewq_o = None
        bufs = ((is0, id0, ts0, td0, hv0, ew0), (is1, id1, ts1, td1, hv1, ew1))
        wid = _wid()
        lo = wid * BKT

        def z65(i, c):
            o = pl.multiple_of(i * 16, 16)
            accT[pl.ds(o, 16)] = jnp.zeros((16,), jnp.float32)
            return c

        def z17(i, c):
            o = pl.multiple_of(i * 16, 16)
            denT[pl.ds(o, 16)] = jnp.zeros((16,), jnp.float32)
            return c

        lax.fori_loop(0, BKT * 65 // 16, z65, 0)
        lax.fori_loop(0, BKT * 17 // 16, z17, 0)
        pltpu.sync_copy(ce_r, cevv)
        pltpu.sync_copy(cnts_r.at[wid], cw)
        cnt = jnp.max(cw[...])
        nch = (cnt + (C - 1)) // C
        ce_b = [cevv[h] for h in range(heads)]
        i16 = _i16()

        def fire(ci, B):
            bis, bid, bts, btd, bhv, bew = B
            off = ci * C
            pltpu.sync_copy(srcq_r.at[wid, pl.ds(off, C)], bis)
            pltpu.sync_copy(dstq_r.at[wid, pl.ds(off, C)], bid)
            if not compute_ew:
                pltpu.sync_copy(ewq_r.at[wid, pl.ds(off, C)], bew)
            for b in range(C // 128):
                sl = pl.ds(b * 128, 128)
                pltpu.async_copy(t_r.at[bis.at[sl]], bts.at[sl], sem)
                pltpu.async_copy(t_r.at[bid.at[sl]], btd.at[sl], sem)
                pltpu.async_copy(h_r.at[bis.at[sl]], bhv.at[sl], sem)

        def wait_for(B):
            bis, bid, bts, btd, bhv, bew = B
            for b in range(C // 128):
                sl = pl.ds(b * 128, 128)
                pltpu.make_async_copy(t_r.at[bis.at[sl]], bts.at[sl], sem).wait()
                pltpu.make_async_copy(t_r.at[bid.at[sl]], btd.at[sl], sem).wait()
                pltpu.make_async_copy(h_r.at[bis.at[sl]], bhv.at[sl], sem).wait()

        def compute(ci, B):
            bis, bid, bts, btd, bhv, bew = B
            off = ci * C

            def jb(j, c2):
                o = pl.multiple_of(j * 16, 16)
                rows = j * 16 + i16
                msk = (off + rows) < cnt
                lr = bid[pl.ds(o, 16)] - lo
                if compute_ew:
                    dacc = None
                    for k in range(3):
                        kf = _f16(16 + k)
                        df = (plsc.load_gather(bts, [rows, kf])
                              - plsc.load_gather(btd, [rows, kf]))
                        dacc = df * df if dacc is None else dacc + df * df
                    ew16 = jnp.exp(dacc * (-1.0 / (2.0 * SIGMA * SIGMA)))
                    bew[pl.ds(o, 16)] = ew16
                else:
                    ew16 = bew[pl.ds(o, 16)]
                lrc = jnp.where(msk, lr, 0)
                sbase = rows * 17
                for h in range(heads):
                    a = (plsc.load_gather(bts, [rows, _f16(h)])
                         + plsc.load_gather(btd, [rows, _f16(h + 8)])
                         + ce_b[h] * ew16)
                    a = jnp.maximum(a, a * 0.2)
                    exm = jnp.where(msk, jnp.exp(a), 0.0)
                    plsc.addupdate_scatter(denT, [lrc * 17 + h], exm)
                    plsc.store_scatter(stageF, [sbase + h], exm)
                return c2

            def jb2(j, c2):
                o = pl.multiple_of(j * 16, 16)
                rows = j * 16 + i16
                msk = (off + rows) < cnt
                lrc = jnp.where(msk, bid[pl.ds(o, 16)] - lo, 0)
                sh = i16 >> LOG2F[fan]
                for e in range(16):
                    lr_e = lrc[e]
                    row_e = jnp.zeros((16,), jnp.int32) + (j * 16 + e)
                    sb = (j * 16 + e) * 17 + (16 * 0 >> LOG2F[fan])
                    for m in range(4):
                        cm = plsc.load_gather(
                            stageF,
                            [(j * 16 + e) * 17 + (16 * m >> LOG2F[fan]) + sh])
                        hm = plsc.load_gather(bhv, [row_e, 16 * m + i16])
                        plsc.addupdate_scatter(
                            accT, [lr_e * 65 + 16 * m + i16], hm * cm)
                return c2

            lax.fori_loop(0, C // 16, jb, 0)
            lax.fori_loop(0, C // 16, jb2, 0)
            if compute_ew:
                pltpu.sync_copy(bew, ewq_o.at[wid, pl.ds(off, C)])

        @pl.when(nch > 0)
        def _prologue():
            fire(0, bufs[0])

        def pair(i, c):
            ci_a = 2 * i
            ci_b = ci_a + 1
            wait_for(bufs[0])

            @pl.when(ci_b < nch)
            def _f1():
                fire(ci_b, bufs[1])

            compute(ci_a, bufs[0])

            @pl.when(ci_b < nch)
            def _p2():
                wait_for(bufs[1])

                @pl.when(ci_b + 1 < nch)
                def _f2():
                    fire(ci_b + 1, bufs[0])

                compute(ci_b, bufs[1])

            return c

        lax.fori_loop(0, (nch + 1) // 2, pair, 0)
        pltpu.sync_copy(accT, acc_o.at[pl.ds(lo * 65, BKT * 65)])
        pltpu.sync_copy(denT, den_o.at[pl.ds(lo * 17, BKT * 17)])

    return pl.kernel(
        body,
        out_type=tuple(out_type),
        mesh=plsc.VectorSubcoreMesh(core_axis_name="c", subcore_axis_name="s"),
        scratch_types=tuple(scratch),
        compiler_params=_SC_PARAMS,
    )


def _prep_body(x_ref, w_ref, a_ref, rx_ref, h_ref, t_ref):
    h = jnp.dot(x_ref[...], w_ref[...], preferred_element_type=jnp.float32)
    h_ref[...] = h
    al = jnp.dot(h, a_ref[...], preferred_element_type=jnp.float32)
    t_ref[...] = jnp.concatenate([al, rx_ref[...]], axis=1)


def _tc_prep(xp, W, A, rxp8):
    BLK = 2048
    return pl.pallas_call(
        _prep_body,
        grid=(NP // BLK,),
        in_specs=[
            pl.BlockSpec((BLK, IN), lambda i: (i, 0)),
            pl.BlockSpec((IN, 64), lambda i: (0, 0)),
            pl.BlockSpec((64, 16), lambda i: (0, 0)),
            pl.BlockSpec((BLK, 8), lambda i: (i, 0)),
        ],
        out_specs=[
            pl.BlockSpec((BLK, 64), lambda i: (i, 0)),
            pl.BlockSpec((BLK, 24), lambda i: (i, 0)),
        ],
        out_shape=[
            jax.ShapeDtypeStruct((NP, 64), jnp.float32),
            jax.ShapeDtypeStruct((NP, 24), jnp.float32),
        ],
    )(xp, W, A, rxp8)


def _recip_body(d_ref, r_ref):
    r_ref[...] = 1.0 / (d_ref[...] + 1e-16)


def _tc_recip(den):
    return pl.pallas_call(
        _recip_body,
        out_shape=jax.ShapeDtypeStruct((NP, 17), jnp.float32),
    )(den)


_BN_SCALE = float((1.0 + 1e-5) ** -0.5)


def _mid_body(acc_ref, rd_ref, b_ref, g_ref, be_ref, w_ref, a_ref,
              h_ref, t_ref):
    blk = acc_ref.shape[0]
    r8 = rd_ref[...][:, :8]
    rexp = jnp.broadcast_to(r8[:, :, None], (blk, 8, 8)).reshape(blk, 64)
    s = acc_ref[...][:, :64] * rexp + b_ref[...][None, :]
    s = s * (g_ref[...] * _BN_SCALE)[None, :] + be_ref[...][None, :]
    s = jnp.where(s > 0, s, jnp.exp(s) - 1.0)
    h = jnp.dot(s, w_ref[...], preferred_element_type=jnp.float32)
    h_ref[...] = h
    t_ref[...] = jnp.dot(h, a_ref[...], preferred_element_type=jnp.float32)


def _tc_mid(acc, rd, b, g, be, W, A):
    BLK = 2048
    return pl.pallas_call(
        _mid_body,
        grid=(NP // BLK,),
        in_specs=[
            pl.BlockSpec((BLK, 65), lambda i: (i, 0)),
            pl.BlockSpec((BLK, 17), lambda i: (i, 0)),
            pl.BlockSpec((64,), lambda i: (0,)),
            pl.BlockSpec((64,), lambda i: (0,)),
            pl.BlockSpec((64,), lambda i: (0,)),
            pl.BlockSpec((64, 64), lambda i: (0, 0)),
            pl.BlockSpec((64, 16), lambda i: (0, 0)),
        ],
        out_specs=[
            pl.BlockSpec((BLK, 64), lambda i: (i, 0)),
            pl.BlockSpec((BLK, 16), lambda i: (i, 0)),
        ],
        out_shape=[
            jax.ShapeDtypeStruct((NP, 64), jnp.float32),
            jax.ShapeDtypeStruct((NP, 16), jnp.float32),
        ],
    )(acc, rd, b, g, be, W, A)


def _tail_body(acc_ref, rd_ref, b_ref, g_ref, be_ref, batch_ref, wc_ref,
               bc_ref, out_ref, pool_ref, cnt_ref):
    i = pl.program_id(0)
    nb = pl.num_programs(0)

    @pl.when(i == 0)
    def _init():
        pool_ref[...] = jnp.zeros_like(pool_ref)
        cnt_ref[...] = jnp.zeros_like(cnt_ref)

    blk = acc_ref.shape[0]
    rexp = jnp.broadcast_to(rd_ref[...][:, :1], (blk, 64))
    s = acc_ref[...][:, :64] * rexp + b_ref[...][None, :]
    s = s * (g_ref[...] * _BN_SCALE)[None, :] + be_ref[...][None, :]
    h = jnp.where(s > 0, s, jnp.exp(s) - 1.0)
    bvec = batch_ref[0, 0]
    onehot = (bvec[None, :] == lax.broadcasted_iota(
        jnp.int32, (NG, bvec.shape[0]), 0)).astype(jnp.float32)
    pool_ref[...] += jnp.dot(onehot, h, preferred_element_type=jnp.float32)
    cnt_ref[...] += jnp.sum(onehot, axis=1, keepdims=True)

    @pl.when(i == nb - 1)
    def _fin():
        gpool = pool_ref[...] / jnp.maximum(cnt_ref[...], 1.0)
        out_ref[...] = jnp.dot(
            gpool, wc_ref[...],
            preferred_element_type=jnp.float32) + bc_ref[...][None, :]


def _tc_tail(acc, rd, b, g, be, batch, Wc, bc):
    BLK = 2000
    return pl.pallas_call(
        _tail_body,
        grid=(N // BLK,),
        in_specs=[
            pl.BlockSpec((BLK, 65), lambda i: (i, 0)),
            pl.BlockSpec((BLK, 17), lambda i: (i, 0)),
            pl.BlockSpec((64,), lambda i: (0,)),
            pl.BlockSpec((64,), lambda i: (0,)),
            pl.BlockSpec((64,), lambda i: (0,)),
            pl.BlockSpec((1, 1, BLK), lambda i: (i, 0, 0)),
            pl.BlockSpec((HID, NC), lambda i: (0, 0)),
            pl.BlockSpec((NC,), lambda i: (0,)),
        ],
        out_specs=pl.BlockSpec((NG, NC), lambda i: (0, 0)),
        out_shape=jax.ShapeDtypeStruct((NG, NC), jnp.float32),
        scratch_shapes=[pltpu.VMEM((NG, HID), jnp.float32),
                        pltpu.VMEM((NG, 1), jnp.float32)],
    )(acc, rd, b, g, be, batch.reshape(N // BLK, 1, BLK), Wc, bc)


def kernel(x, edge_index, raw_x, batch, W1, att_src1, att_dst1, We1,
           att_edge1, b1, g1, be1, W2, att_src2, att_dst2, We2, att_edge2,
           b2, g2, be2, Wc, bc):
    src = edge_index[0]
    dst = edge_index[1]
    srcq, dstq, cnts = _pre_call()(src, dst)

    xp = jnp.pad(x, ((0, NP - N), (0, 0)))
    rxp8 = jnp.pad(raw_x, ((0, NP - N), (0, 5)))

    eye8 = jnp.eye(8, dtype=jnp.float32)
    As1 = (eye8[:, None, :] * att_src1[:, :, None]).reshape(64, 8)
    Ad1 = (eye8[:, None, :] * att_dst1[:, :, None]).reshape(64, 8)
    A1 = jnp.concatenate([As1, Ad1], axis=1)
    ce1 = (We1.reshape(HEADS, HD) * att_edge1).sum(-1)
    cev1 = jnp.repeat(ce1[:, None], 16, axis=1)

    A2 = jnp.zeros((64, 16), jnp.float32)
    A2 = A2.at[:, 0].set(att_src2[0]).at[:, 8].set(att_dst2[0])
    ce2 = (We2[0] * att_edge2[0]).sum()
    cev2 = jnp.full((1, 16), ce2, jnp.float32)

    h1, T1 = _tc_prep(xp, W1, A1, rxp8)
    accf1, denf1, ewq = _layer_call(HEADS, 24, True)(
        srcq, dstq, cnts, T1, h1, cev1)
    rden1 = _tc_recip(denf1.reshape(NP, 17))
    h2, T2 = _tc_mid(accf1.reshape(NP, 65), rden1, b1, g1, be1, W2, A2)
    accf2, denf2 = _layer_call(1, 16, False)(
        srcq, dstq, cnts, T2, h2, cev2, ewq)
    rden2 = _tc_recip(denf2.reshape(NP, 17))
    return _tc_tail(accf2.reshape(NP, 65), rden2, b2, g2, be2, batch, Wc, bc)
